# baseline (speedup 1.0000x reference)
import functools

import jax
import jax.numpy as jnp
from jax import lax
from jax.experimental import pallas as pl
from jax.experimental.pallas import tpu as pltpu
from jax.experimental.pallas import tpu_sc as plsc

_B = 16384
_D = 64
_NEG = 5
_CB = 2048


def _sc_gather(idx, table):
    info = plsc.get_sparse_core_info()
    nw = info.num_cores * info.num_subcores
    b_per_w = _B // nw
    mesh = plsc.VectorSubcoreMesh(core_axis_name="c", subcore_axis_name="s")

    @functools.partial(
        pl.kernel,
        mesh=mesh,
        out_type=jax.ShapeDtypeStruct((_B, _D), jnp.float32),
        scratch_types=[
            pltpu.VMEM((b_per_w,), jnp.int32),
            pltpu.VMEM((b_per_w, _D), jnp.float32),
            pltpu.SemaphoreType.DMA,
        ],
        compiler_params=pltpu.CompilerParams(use_tc_tiling_on_sc=False),
    )
    def k(idx_hbm, table_hbm, out_hbm, idx_v, rows_v, sem):
        wid = lax.axis_index("s") * info.num_cores + lax.axis_index("c")
        base = wid * b_per_w
        pltpu.sync_copy(idx_hbm.at[pl.ds(base, b_per_w)], idx_v)
        pltpu.async_copy(table_hbm.at[idx_v], rows_v, sem).wait()
        pltpu.sync_copy(rows_v, out_hbm.at[pl.ds(base, b_per_w)])

    return k(idx, table)


def _tc_loss_body(emb_ref, pos_ref, neg_ref, out_ref):
    i = pl.program_id(0)
    e = emb_ref[...]
    p = pos_ref[...]
    nf = neg_ref[...]
    s = jnp.sum(p * e, axis=1, keepdims=True)
    s = jnp.clip(s, -10.0, 10.0)
    loss_pos = jnp.log1p(jnp.exp(-s))
    ns = jnp.sum(nf * e[:, None, :], axis=2)
    ns = jnp.clip(ns, -10.0, 10.0)
    loss_neg = jnp.log1p(jnp.exp(ns))
    part = (jnp.sum(loss_pos) + jnp.sum(loss_neg)) * (1.0 / _B)

    @pl.when(i == 0)
    def _():
        out_ref[0, 0] = 0.0

    out_ref[0, 0] += part


def _tc_loss(emb, pos, neg):
    grid = _B // _CB
    return pl.pallas_call(
        _tc_loss_body,
        grid=(grid,),
        in_specs=[
            pl.BlockSpec((_CB, _D), lambda i: (i, 0)),
            pl.BlockSpec((_CB, _D), lambda i: (i, 0)),
            pl.BlockSpec((_CB, _NEG, _D), lambda i: (i, 0, 0)),
        ],
        out_specs=pl.BlockSpec(memory_space=pltpu.SMEM),
        out_shape=jax.ShapeDtypeStruct((1, 1), jnp.float32),
    )(emb, pos, neg)


def kernel(words, pos_features, neg_features, u_embeddings):
    emb = _sc_gather(words, u_embeddings)
    loss = _tc_loss(emb, pos_features, neg_features)
    return jnp.reshape(loss, ())


# speedup vs baseline: 1.5880x; 1.5880x over previous
import functools

import jax
import jax.numpy as jnp
from jax import lax
from jax.experimental import pallas as pl
from jax.experimental.pallas import tpu as pltpu
from jax.experimental.pallas import tpu_sc as plsc

_B = 16384
_D = 64
_NEG = 5
_CB = 2048


def _sc_gather(idx, table):
    info = plsc.get_sparse_core_info()
    nw = info.num_cores * info.num_subcores
    b_per_w = _B // nw
    mesh = plsc.VectorSubcoreMesh(core_axis_name="c", subcore_axis_name="s")

    @functools.partial(
        pl.kernel,
        mesh=mesh,
        out_type=jax.ShapeDtypeStruct((_B, _D), jnp.float32),
        scratch_types=[
            pltpu.VMEM((b_per_w,), jnp.int32),
            pltpu.VMEM((b_per_w, _D), jnp.float32),
            pltpu.SemaphoreType.DMA,
            pltpu.SemaphoreType.DMA,
        ],
        compiler_params=pltpu.CompilerParams(use_tc_tiling_on_sc=True),
    )
    def k(idx_hbm, table_hbm, out_hbm, idx_v, rows_v, isem, sem):
        wid = lax.axis_index("s") * info.num_cores + lax.axis_index("c")
        base = wid * b_per_w
        pltpu.async_copy(idx_hbm.at[pl.ds(base, b_per_w)], idx_v, isem).wait()

        def fire(j, _):
            vec = idx_v[pl.ds(j * 16, 16)]
            for t in range(16):
                row = vec[t]
                pltpu.async_copy(
                    table_hbm.at[pl.ds(row, 1)],
                    rows_v.at[pl.ds(j * 16 + t, 1)],
                    sem,
                )
            return 0

        lax.fori_loop(0, b_per_w // 16, fire, 0)
        pltpu.make_async_copy(
            table_hbm.at[pl.ds(0, b_per_w)], rows_v, sem
        ).wait()
        pltpu.async_copy(rows_v, out_hbm.at[pl.ds(base, b_per_w)], isem).wait()

    return k(idx, table)


def _tc_loss_body(emb_ref, pos_ref, neg_ref, out_ref):
    i = pl.program_id(0)
    e = emb_ref[...]
    p = pos_ref[...]
    nf = neg_ref[...]
    s = jnp.sum(p * e, axis=1, keepdims=True)
    s = jnp.clip(s, -10.0, 10.0)
    loss_pos = jnp.log1p(jnp.exp(-s))
    ns = jnp.sum(nf * e[:, None, :], axis=2)
    ns = jnp.clip(ns, -10.0, 10.0)
    loss_neg = jnp.log1p(jnp.exp(ns))
    part = (jnp.sum(loss_pos) + jnp.sum(loss_neg)) * (1.0 / _B)

    @pl.when(i == 0)
    def _():
        out_ref[0, 0] = 0.0

    out_ref[0, 0] += part


def _tc_loss(emb, pos, neg):
    grid = _B // _CB
    return pl.pallas_call(
        _tc_loss_body,
        grid=(grid,),
        in_specs=[
            pl.BlockSpec((_CB, _D), lambda i: (i, 0)),
            pl.BlockSpec((_CB, _D), lambda i: (i, 0)),
            pl.BlockSpec((_CB, _NEG, _D), lambda i: (i, 0, 0)),
        ],
        out_specs=pl.BlockSpec(memory_space=pltpu.SMEM),
        out_shape=jax.ShapeDtypeStruct((1, 1), jnp.float32),
    )(emb, pos, neg)


def kernel(words, pos_features, neg_features, u_embeddings):
    emb = _sc_gather(words, u_embeddings)
    loss = _tc_loss(emb, pos_features, neg_features)
    return jnp.reshape(loss, ())


# speedup vs baseline: 1.9043x; 1.1992x over previous
import functools

import jax
import jax.numpy as jnp
from jax import lax
from jax.experimental import pallas as pl
from jax.experimental.pallas import tpu as pltpu
from jax.experimental.pallas import tpu_sc as plsc

_B = 16384
_D = 64
_NEG = 5
_CB = 1024


def _sc_gather(idx, table):
    info = plsc.get_sparse_core_info()
    nw = info.num_cores * info.num_subcores
    b_per_w = _B // nw
    mesh = plsc.VectorSubcoreMesh(core_axis_name="c", subcore_axis_name="s")

    @functools.partial(
        pl.kernel,
        mesh=mesh,
        out_type=jax.ShapeDtypeStruct((_B, _D), jnp.float32),
        scratch_types=[
            pltpu.VMEM((b_per_w,), jnp.int32),
            pltpu.VMEM((b_per_w, _D), jnp.float32),
            pltpu.SemaphoreType.DMA,
            pltpu.SemaphoreType.DMA,
        ],
        compiler_params=pltpu.CompilerParams(use_tc_tiling_on_sc=True),
    )
    def k(idx_hbm, table_hbm, out_hbm, idx_v, rows_v, isem, sem):
        wid = lax.axis_index("s") * info.num_cores + lax.axis_index("c")
        base = wid * b_per_w
        pltpu.async_copy(idx_hbm.at[pl.ds(base, b_per_w)], idx_v, isem).wait()

        def fire(j, _):
            vec = idx_v[pl.ds(j * 16, 16)]
            for t in range(16):
                row = vec[t]
                pltpu.async_copy(
                    table_hbm.at[pl.ds(row, 1)],
                    rows_v.at[pl.ds(j * 16 + t, 1)],
                    sem,
                )
            return 0

        lax.fori_loop(0, b_per_w // 16, fire, 0)
        pltpu.make_async_copy(
            table_hbm.at[pl.ds(0, b_per_w)], rows_v, sem
        ).wait()
        pltpu.async_copy(rows_v, out_hbm.at[pl.ds(base, b_per_w)], isem).wait()

    return k(idx, table)


def _tc_loss_body(emb_ref, pos_ref, neg_ref, out_ref):
    i = pl.program_id(0)
    et = jnp.transpose(emb_ref[...])
    s = jnp.sum(pos_ref[...] * et, axis=0, keepdims=True)
    s = jnp.clip(s, -10.0, 10.0)
    acc = jnp.log1p(jnp.exp(-s))
    for kk in range(_NEG):
        ns = jnp.sum(neg_ref[kk] * et, axis=0, keepdims=True)
        ns = jnp.clip(ns, -10.0, 10.0)
        acc += jnp.log1p(jnp.exp(ns))
    part = jnp.sum(acc) * (1.0 / _B)

    @pl.when(i == 0)
    def _():
        out_ref[0, 0] = 0.0

    out_ref[0, 0] += part


def _tc_loss(emb, pos_t, neg_t):
    grid = _B // _CB
    return pl.pallas_call(
        _tc_loss_body,
        grid=(grid,),
        in_specs=[
            pl.BlockSpec((_CB, _D), lambda i: (i, 0)),
            pl.BlockSpec((_D, _CB), lambda i: (0, i)),
            pl.BlockSpec((_NEG, _D, _CB), lambda i: (0, 0, i)),
        ],
        out_specs=pl.BlockSpec(memory_space=pltpu.SMEM),
        out_shape=jax.ShapeDtypeStruct((1, 1), jnp.float32),
    )(emb, pos_t, neg_t)


def kernel(words, pos_features, neg_features, u_embeddings):
    pos_t = pos_features.T
    neg_t = jnp.transpose(neg_features, (1, 2, 0))
    emb = _sc_gather(words, u_embeddings)
    loss = _tc_loss(emb, pos_t, neg_t)
    return jnp.reshape(loss, ())


# speedup vs baseline: 1.9056x; 1.0007x over previous
import functools

import jax
import jax.numpy as jnp
from jax import lax
from jax.experimental import pallas as pl
from jax.experimental.pallas import tpu as pltpu
from jax.experimental.pallas import tpu_sc as plsc

_B = 16384
_D = 64
_NEG = 5
_CB = 1024


def _sc_gather(idx, table):
    info = plsc.get_sparse_core_info()
    nw = info.num_cores * info.num_subcores
    b_per_w = _B // nw
    mesh = plsc.VectorSubcoreMesh(core_axis_name="c", subcore_axis_name="s")

    @functools.partial(
        pl.kernel,
        mesh=mesh,
        out_type=jax.ShapeDtypeStruct((_B, _D), jnp.float32),
        scratch_types=[
            pltpu.VMEM((b_per_w,), jnp.int32),
            pltpu.VMEM((b_per_w, _D), jnp.float32),
            pltpu.SemaphoreType.DMA,
            pltpu.SemaphoreType.DMA,
        ],
        compiler_params=pltpu.CompilerParams(use_tc_tiling_on_sc=True),
    )
    def k(idx_hbm, table_hbm, out_hbm, idx_v, rows_v, isem, sem):
        wid = lax.axis_index("s") * info.num_cores + lax.axis_index("c")
        base = wid * b_per_w
        pltpu.async_copy(idx_hbm.at[pl.ds(base, b_per_w)], idx_v, isem).wait()

        def fire(j, _):
            vec = idx_v[pl.ds(j * 16, 16)]
            for t in range(16):
                row = vec[t]
                pltpu.async_copy(
                    table_hbm.at[pl.ds(row, 1)],
                    rows_v.at[pl.ds(j * 16 + t, 1)],
                    sem,
                )
            return 0

        lax.fori_loop(0, b_per_w // 16, fire, 0)
        pltpu.make_async_copy(
            table_hbm.at[pl.ds(0, b_per_w)], rows_v, sem
        ).wait()
        pltpu.async_copy(rows_v, out_hbm.at[pl.ds(base, b_per_w)], isem).wait()

    return k(idx, table)


_V = 1000000
_TCB = 4096


def _tt_body(tt_ref, out_ref):
    out_ref[...] = jnp.transpose(tt_ref[...])


def _tc_transpose(table_t):
    grid = (_V + _TCB - 1) // _TCB
    return pl.pallas_call(
        _tt_body,
        grid=(grid,),
        in_specs=[pl.BlockSpec((_D, _TCB), lambda i: (0, i))],
        out_specs=pl.BlockSpec((_TCB, _D), lambda i: (i, 0)),
        out_shape=jax.ShapeDtypeStruct((_V, _D), jnp.float32),
    )(table_t)


def _tc_loss_body(emb_ref, pos_ref, neg_ref, out_ref):
    i = pl.program_id(0)
    et = jnp.transpose(emb_ref[...])
    s = jnp.sum(pos_ref[...] * et, axis=0, keepdims=True)
    s = jnp.clip(s, -10.0, 10.0)
    acc = jnp.log1p(jnp.exp(-s))
    for kk in range(_NEG):
        ns = jnp.sum(neg_ref[kk] * et, axis=0, keepdims=True)
        ns = jnp.clip(ns, -10.0, 10.0)
        acc += jnp.log1p(jnp.exp(ns))
    part = jnp.sum(acc) * (1.0 / _B)

    @pl.when(i == 0)
    def _():
        out_ref[0, 0] = 0.0

    out_ref[0, 0] += part


def _tc_loss(emb, pos_t, neg_t):
    grid = _B // _CB
    return pl.pallas_call(
        _tc_loss_body,
        grid=(grid,),
        in_specs=[
            pl.BlockSpec((_CB, _D), lambda i: (i, 0)),
            pl.BlockSpec((_D, _CB), lambda i: (0, i)),
            pl.BlockSpec((_NEG, _D, _CB), lambda i: (0, 0, i)),
        ],
        out_specs=pl.BlockSpec(memory_space=pltpu.SMEM),
        out_shape=jax.ShapeDtypeStruct((1, 1), jnp.float32),
    )(emb, pos_t, neg_t)


def kernel(words, pos_features, neg_features, u_embeddings):
    pos_t = pos_features.T
    neg_t = jnp.transpose(neg_features, (1, 2, 0))
    table_rm = _tc_transpose(u_embeddings.T)
    emb = _sc_gather(words, table_rm)
    loss = _tc_loss(emb, pos_t, neg_t)
    return jnp.reshape(loss, ())
